# Initial kernel scaffold; baseline (speedup 1.0000x reference)
import jax
import jax.numpy as jnp
from jax.experimental import pallas as pl


def kernel(x, edge_index, batch, W1, b1, W2, b2, gn_w, gn_b, gn_ms, ln_w, ln_b, fcW, fcb):
    raise NotImplementedError("write your pallas kernel here")


# speedup vs baseline: 16.8637x; 16.8637x over previous
import functools

import jax
import jax.numpy as jnp
from jax import lax
from jax.experimental import pallas as pl
from jax.experimental.pallas import tpu as pltpu
from jax.experimental.pallas import tpu_sc as plsc

_N = 10000
_E = 320000
_DIN = 128
_H = 64
_DOUT = 64
_B = 64
_EPS = 1e-5

_NC = 2
_NS = 16
_NW = _NC * _NS
_EPT = _E // _NW
_K = 80
_CH = _EPT // _K
_NP = 10240
_RPT = _NP // _NS
_W = 128


def _mesh():
    return plsc.VectorSubcoreMesh(core_axis_name="c", subcore_axis_name="s",
                                  num_cores=_NC, num_subcores=_NS)


def _deg_body(dstr_hbm, ones_hbm, zeros_hbm, out_hbm, dst_v, ones_v, acc_sh):
    c = lax.axis_index("c")
    s = lax.axis_index("s")
    wid = c * _NS + s
    pltpu.sync_copy(zeros_hbm.at[pl.ds(s * _RPT, _RPT)],
                    acc_sh.at[pl.ds(s * _RPT, _RPT)])
    pltpu.sync_copy(dstr_hbm.at[wid], dst_v)
    pltpu.sync_copy(ones_hbm, ones_v)
    plsc.subcore_barrier()

    def chunk(j, carry):
        pltpu.sync_copy(ones_v, acc_sh.at[dst_v.at[j]], add=True)
        return carry

    lax.fori_loop(0, _CH, chunk, 0)
    plsc.subcore_barrier()
    pltpu.sync_copy(acc_sh.at[pl.ds(s * _RPT, _RPT)],
                    out_hbm.at[c, pl.ds(s * _RPT, _RPT)])


@functools.cache
def _deg_kernel():
    return pl.kernel(
        _deg_body,
        out_type=jax.ShapeDtypeStruct((_NC, _NP), jnp.float32),
        mesh=_mesh(),
        scratch_types=[
            pltpu.VMEM((_CH, _K), jnp.int32),
            pltpu.VMEM((_K,), jnp.float32),
            pltpu.VMEM_SHARED((_NP,), jnp.float32),
        ],
    )


def _edge_body(srcr_hbm, dstr_hbm, hs_hbm, zeros_hbm, out_hbm,
               src_v, dst_v, rows_v, acc_sh, sem):
    c = lax.axis_index("c")
    s = lax.axis_index("s")
    wid = c * _NS + s
    pltpu.sync_copy(zeros_hbm.at[pl.ds(s * _RPT, _RPT)],
                    acc_sh.at[pl.ds(s * _RPT, _RPT)])
    pltpu.sync_copy(srcr_hbm.at[wid], src_v)
    pltpu.sync_copy(dstr_hbm.at[wid], dst_v)
    plsc.subcore_barrier()

    def chunk(j, carry):
        pltpu.async_copy(hs_hbm.at[src_v.at[j]], rows_v, sem).wait()
        pltpu.sync_copy(rows_v, acc_sh.at[dst_v.at[j]], add=True)
        return carry

    lax.fori_loop(0, _CH, chunk, 0)
    plsc.subcore_barrier()
    pltpu.sync_copy(acc_sh.at[pl.ds(s * _RPT, _RPT)],
                    out_hbm.at[c, pl.ds(s * _RPT, _RPT)])


@functools.cache
def _edge_kernel():
    return pl.kernel(
        _edge_body,
        out_type=jax.ShapeDtypeStruct((_NC, _NP, _W), jnp.float32),
        mesh=_mesh(),
        scratch_types=[
            pltpu.VMEM((_CH, _K), jnp.int32),
            pltpu.VMEM((_CH, _K), jnp.int32),
            pltpu.VMEM((_K, _W), jnp.float32),
            pltpu.VMEM_SHARED((_NP, _W), jnp.float32),
            pltpu.SemaphoreType.DMA,
        ],
    )


def _prep_body(degp_ref, x_ref, w1_ref, dinv_ref, hs_ref):
    deg = (degp_ref[0, pl.ds(0, _N)]
           + degp_ref[1, pl.ds(0, _N)] + 1.0)
    dinv = lax.rsqrt(deg).reshape(_N, 1)
    dinv_ref[...] = dinv
    hs_ref[...] = jnp.dot(x_ref[...], w1_ref[...],
                          preferred_element_type=jnp.float32) * dinv


def _prep_call(degp, x, W1p):
    return pl.pallas_call(
        _prep_body,
        out_shape=(jax.ShapeDtypeStruct((_N, 1), jnp.float32),
                   jax.ShapeDtypeStruct((_N, _W), jnp.float32)),
    )(degp, x, W1p)


def _norm_block(accp_ref, hs, dinv, b, gnw, gnb, gnms, lnw, lnb):
    accp = (accp_ref[0, pl.ds(0, _N), pl.ds(0, _H)]
            + accp_ref[1, pl.ds(0, _N), pl.ds(0, _H)])
    acc = (accp + hs) * dinv + b
    h = jnp.maximum(acc, 0.0)
    mean = jnp.mean(h, axis=0, keepdims=True)
    o = h - gnms * mean
    var = jnp.mean(o * o, axis=0, keepdims=True)
    o = o * lax.rsqrt(var + _EPS) * gnw + gnb
    mu = jnp.mean(o, axis=1, keepdims=True)
    v = jnp.mean((o - mu) ** 2, axis=1, keepdims=True)
    return (o - mu) * lax.rsqrt(v + _EPS) * lnw + lnb


def _mid_body(accp_ref, hs_ref, dinv_ref, b1_ref, gnw_ref, gnb_ref,
              gnms_ref, lnw_ref, lnb_ref, w2_ref, hs2_ref):
    dinv = dinv_ref[...]
    o = _norm_block(accp_ref, hs_ref[pl.ds(0, _N), pl.ds(0, _H)], dinv,
                    b1_ref[...], gnw_ref[...], gnb_ref[...], gnms_ref[...],
                    lnw_ref[...], lnb_ref[...])
    hs2_ref[...] = jnp.dot(o, w2_ref[...],
                           preferred_element_type=jnp.float32) * dinv


def _mid_call(accp, hs, dinv, b1, gnw, gnb, gnms, lnw, lnb, W2p):
    return pl.pallas_call(
        _mid_body,
        out_shape=jax.ShapeDtypeStruct((_N, _W), jnp.float32),
    )(accp, hs, dinv, b1, gnw, gnb, gnms, lnw, lnb, W2p)


def _final_body(accp_ref, hs_ref, dinv_ref, b2_ref, gnw_ref, gnb_ref,
                gnms_ref, lnw_ref, lnb_ref, bat_ref, fcw_ref, fcb_ref,
                out_ref):
    o = _norm_block(accp_ref, hs_ref[pl.ds(0, _N), pl.ds(0, _H)],
                    dinv_ref[...], b2_ref[...], gnw_ref[...], gnb_ref[...],
                    gnms_ref[...], lnw_ref[...], lnb_ref[...])
    bat = bat_ref[...]
    rows = lax.broadcasted_iota(jnp.int32, (_B, 1), 0)
    neg = jnp.float32(-jnp.inf)

    def seg(b, pooled):
        mx = jnp.max(jnp.where(bat == b, o, neg), axis=0, keepdims=True)
        return jnp.where(rows == b, mx, pooled)

    pooled = lax.fori_loop(0, _B, seg,
                           jnp.full((_B, _H), neg, jnp.float32))
    out_ref[...] = jnp.dot(pooled, fcw_ref[...],
                           preferred_element_type=jnp.float32) + fcb_ref[...]


def _final_call(accp, hs, dinv, b2, gnw, gnb, gnms, lnw, lnb, bat, fcW, fcb):
    return pl.pallas_call(
        _final_body,
        out_shape=jax.ShapeDtypeStruct((_B, _DOUT), jnp.float32),
    )(accp, hs, dinv, b2, gnw, gnb, gnms, lnw, lnb, bat, fcW, fcb)


def kernel(x, edge_index, batch, W1, b1, W2, b2, gn_w, gn_b, gn_ms,
           ln_w, ln_b, fcW, fcb):
    src_r = edge_index[0].reshape(_NW, _CH, _K)
    dst_r = edge_index[1].reshape(_NW, _CH, _K)
    ones_d = jnp.ones((_K,), jnp.float32)
    z_d = jnp.zeros((_NP,), jnp.float32)
    z_h = jnp.zeros((_NP, _W), jnp.float32)
    W1p = jnp.pad(W1, ((0, 0), (0, _W - _H)))
    W2p = jnp.pad(W2, ((0, 0), (0, _W - _H)))

    degp = _deg_kernel()(dst_r, ones_d, z_d)
    dinv, hs1 = _prep_call(degp, x, W1p)
    acc1 = _edge_kernel()(src_r, dst_r, hs1, z_h)
    hs2 = _mid_call(acc1, hs1, dinv, b1.reshape(1, _H),
                    gn_w.reshape(1, _H), gn_b.reshape(1, _H),
                    gn_ms.reshape(1, _H), ln_w.reshape(1, _H),
                    ln_b.reshape(1, _H), W2p)
    acc2 = _edge_kernel()(src_r, dst_r, hs2, z_h)
    out = _final_call(acc2, hs2, dinv, b2.reshape(1, _H),
                      gn_w.reshape(1, _H), gn_b.reshape(1, _H),
                      gn_ms.reshape(1, _H), ln_w.reshape(1, _H),
                      ln_b.reshape(1, _H), batch.reshape(_N, 1),
                      fcW, fcb.reshape(1, _DOUT))
    return out


# speedup vs baseline: 30.3948x; 1.8024x over previous
import functools

import jax
import jax.numpy as jnp
from jax import lax
from jax.experimental import pallas as pl
from jax.experimental.pallas import tpu as pltpu
from jax.experimental.pallas import tpu_sc as plsc

_N = 10000
_E = 320000
_DIN = 128
_H = 64
_DOUT = 64
_B = 64
_EPS = 1e-5

_NC = 2
_NS = 16
_NW = _NC * _NS
_EPT = _E // _NW
_K = 80
_CH = _EPT // _K
_NP = 10240
_RPT = _NP // _NS
_W = 128


def _mesh():
    return plsc.VectorSubcoreMesh(core_axis_name="c", subcore_axis_name="s",
                                  num_cores=_NC, num_subcores=_NS)


def _deg_body(dstr_hbm, ones_hbm, zeros_hbm, out_hbm, dst_v, ones_v, acc_sh,
              ssem):
    c = lax.axis_index("c")
    s = lax.axis_index("s")
    wid = c * _NS + s
    pltpu.sync_copy(zeros_hbm, acc_sh.at[pl.ds(s * _RPT, _RPT)])
    pltpu.sync_copy(dstr_hbm.at[wid], dst_v)
    pltpu.sync_copy(ones_hbm, ones_v)
    plsc.subcore_barrier()

    def fire(j, carry):
        pltpu.async_copy(ones_v, acc_sh.at[dst_v.at[j]], ssem, add=True)
        return carry

    lax.fori_loop(0, _CH, fire, 0)

    def drain(j, carry):
        pltpu.make_async_copy(ones_v, acc_sh.at[dst_v.at[j]], ssem).wait()
        return carry

    lax.fori_loop(0, _CH, drain, 0)
    plsc.subcore_barrier()
    pltpu.sync_copy(acc_sh.at[pl.ds(s * _RPT, _RPT)],
                    out_hbm.at[c, pl.ds(s * _RPT, _RPT)])


@functools.cache
def _deg_kernel():
    return pl.kernel(
        _deg_body,
        out_type=jax.ShapeDtypeStruct((_NC, _NP), jnp.float32),
        mesh=_mesh(),
        scratch_types=[
            pltpu.VMEM((_CH, _K), jnp.int32),
            pltpu.VMEM((_K,), jnp.float32),
            pltpu.VMEM_SHARED((_NP,), jnp.float32),
            pltpu.SemaphoreType.DMA,
        ],
    )


def _edge_body(srcf_hbm, dstr_hbm, hs_hbm, zeros_hbm, out_hbm,
               src_v, dst_v, buf0, buf1, acc_sh, sem0, sem1):
    c = lax.axis_index("c")
    s = lax.axis_index("s")
    wid = c * _NS + s
    pltpu.sync_copy(zeros_hbm, acc_sh.at[pl.ds(s * _RPT, _RPT)])
    pltpu.sync_copy(srcf_hbm.at[wid], src_v)
    pltpu.sync_copy(dstr_hbm.at[wid], dst_v)
    plsc.subcore_barrier()

    def g(j, buf, sem):
        pltpu.async_copy(hs_hbm.at[src_v.at[pl.ds(j * _K, _K)]], buf, sem)

    def gw(buf, sem):
        pltpu.make_async_copy(hs_hbm.at[src_v.at[pl.ds(0, _K)]],
                              buf, sem).wait()

    g(0, buf0, sem0)
    g(1, buf1, sem1)

    def chunk(i, carry):
        a = 2 * i
        gw(buf0, sem0)
        pltpu.sync_copy(buf0, acc_sh.at[dst_v.at[a]], add=True)
        g(a + 2, buf0, sem0)
        gw(buf1, sem1)
        pltpu.sync_copy(buf1, acc_sh.at[dst_v.at[a + 1]], add=True)
        nxt = jnp.minimum(a + 3, _CH - 1)
        g(nxt, buf1, sem1)
        return carry

    lax.fori_loop(0, (_CH - 1) // 2, chunk, 0)
    gw(buf0, sem0)
    pltpu.sync_copy(buf0, acc_sh.at[dst_v.at[_CH - 1]], add=True)
    gw(buf1, sem1)
    plsc.subcore_barrier()
    pltpu.sync_copy(acc_sh.at[pl.ds(s * _RPT, _RPT)],
                    out_hbm.at[c, pl.ds(s * _RPT, _RPT)])


@functools.cache
def _edge_kernel():
    return pl.kernel(
        _edge_body,
        out_type=jax.ShapeDtypeStruct((_NC, _NP, _W), jnp.float32),
        mesh=_mesh(),
        scratch_types=[
            pltpu.VMEM((_EPT,), jnp.int32),
            pltpu.VMEM((_CH, _K), jnp.int32),
            pltpu.VMEM((_K, _W), jnp.float32),
            pltpu.VMEM((_K, _W), jnp.float32),
            pltpu.VMEM_SHARED((_NP, _W), jnp.float32),
            pltpu.SemaphoreType.DMA,
            pltpu.SemaphoreType.DMA,
        ],
    )


def _prep_body(degp_ref, x_ref, w1_ref, bat_ref, dinv_ref, hs_ref, offs_ref):
    deg = (degp_ref[0, pl.ds(0, _N)]
           + degp_ref[1, pl.ds(0, _N)] + 1.0)
    dinv = lax.rsqrt(deg).reshape(_N, 1)
    dinv_ref[...] = dinv
    hs_ref[...] = jnp.dot(x_ref[...], w1_ref[...],
                          preferred_element_type=jnp.float32) * dinv
    bat = bat_ref[...].reshape(_N, 1)
    iot = lax.broadcasted_iota(jnp.int32, (1, _B), 1)
    lo = jnp.sum((bat < iot).astype(jnp.int32), axis=0, keepdims=True)
    hi = jnp.sum((bat <= iot).astype(jnp.int32), axis=0, keepdims=True)
    offs_ref[...] = jnp.concatenate([lo, hi], axis=0)


def _prep_call(degp, x, W1p, batch):
    return pl.pallas_call(
        _prep_body,
        out_shape=(jax.ShapeDtypeStruct((_N, 1), jnp.float32),
                   jax.ShapeDtypeStruct((_N, _W), jnp.float32),
                   jax.ShapeDtypeStruct((2, _B), jnp.int32)),
    )(degp, x, W1p, batch)


def _norm_block(accp_ref, hs, dinv, b, gnw, gnb, gnms, lnw, lnb):
    accp = (accp_ref[0, pl.ds(0, _N), pl.ds(0, _H)]
            + accp_ref[1, pl.ds(0, _N), pl.ds(0, _H)])
    acc = (accp + hs) * dinv + b
    h = jnp.maximum(acc, 0.0)
    mean = jnp.mean(h, axis=0, keepdims=True)
    o = h - gnms * mean
    var = jnp.mean(o * o, axis=0, keepdims=True)
    o = o * lax.rsqrt(var + _EPS) * gnw + gnb
    mu = jnp.mean(o, axis=1, keepdims=True)
    v = jnp.mean((o - mu) ** 2, axis=1, keepdims=True)
    return (o - mu) * lax.rsqrt(v + _EPS) * lnw + lnb


def _mid_body(accp_ref, hs_ref, dinv_ref, b1_ref, gnw_ref, gnb_ref,
              gnms_ref, lnw_ref, lnb_ref, w2_ref, hs2_ref):
    dinv = dinv_ref[...]
    o = _norm_block(accp_ref, hs_ref[pl.ds(0, _N), pl.ds(0, _H)], dinv,
                    b1_ref[...], gnw_ref[...], gnb_ref[...], gnms_ref[...],
                    lnw_ref[...], lnb_ref[...])
    hs2_ref[...] = jnp.dot(o, w2_ref[...],
                           preferred_element_type=jnp.float32) * dinv


def _mid_call(accp, hs, dinv, b1, gnw, gnb, gnms, lnw, lnb, W2p):
    return pl.pallas_call(
        _mid_body,
        out_shape=jax.ShapeDtypeStruct((_N, _W), jnp.float32),
    )(accp, hs, dinv, b1, gnw, gnb, gnms, lnw, lnb, W2p)


_SW = 256


def _final_body(accp_ref, hs_ref, dinv_ref, b2_ref, gnw_ref, gnb_ref,
                gnms_ref, lnw_ref, lnb_ref, bat_ref, offs_ref, fcw_ref,
                fcb_ref, out_ref, o_scr):
    o = _norm_block(accp_ref, hs_ref[pl.ds(0, _N), pl.ds(0, _H)],
                    dinv_ref[...], b2_ref[...], gnw_ref[...], gnb_ref[...],
                    gnms_ref[...], lnw_ref[...], lnb_ref[...])
    o_scr[pl.ds(0, _N), :] = o
    rows = lax.broadcasted_iota(jnp.int32, (_B, 1), 0)
    neg = jnp.float32(-jnp.inf)

    def seg(b, pooled):
        base = (offs_ref[0, b] // 128) * 128
        nb = offs_ref[1, b] - base

        def cond(carry):
            i, _ = carry
            return i * _SW < nb

        def wbody(carry):
            i, acc = carry
            start = jnp.minimum(base + i * _SW, _NP - _SW)
            start = pl.multiple_of(start, 128)
            w = o_scr[pl.ds(start, _SW), :]
            m = bat_ref[pl.ds(start, _SW)].reshape(_SW, 1) == b
            mx = jnp.max(jnp.where(m, w, neg), axis=0, keepdims=True)
            return i + 1, jnp.maximum(acc, mx)

        _, mx = lax.while_loop(cond, wbody,
                               (0, jnp.full((1, _H), neg, jnp.float32)))
        return jnp.where(rows == b, mx, pooled)

    pooled = lax.fori_loop(0, _B, seg,
                           jnp.full((_B, _H), neg, jnp.float32))
    out_ref[...] = jnp.dot(pooled, fcw_ref[...],
                           preferred_element_type=jnp.float32) + fcb_ref[...]


def _final_call(accp, hs, dinv, b2, gnw, gnb, gnms, lnw, lnb, bat, offs,
                fcW, fcb):
    specs = [pl.BlockSpec()] * 13
    specs[10] = pl.BlockSpec(memory_space=pltpu.SMEM)
    return pl.pallas_call(
        _final_body,
        out_shape=jax.ShapeDtypeStruct((_B, _DOUT), jnp.float32),
        in_specs=specs,
        scratch_shapes=[pltpu.VMEM((_NP, _H), jnp.float32)],
    )(accp, hs, dinv, b2, gnw, gnb, gnms, lnw, lnb, bat, offs, fcW, fcb)


def kernel(x, edge_index, batch, W1, b1, W2, b2, gn_w, gn_b, gn_ms,
           ln_w, ln_b, fcW, fcb):
    src_f = edge_index[0].reshape(_NW, _EPT)
    dst_r = edge_index[1].reshape(_NW, _CH, _K)
    ones_d = jnp.ones((_K,), jnp.float32)
    z_d = jnp.zeros((_RPT,), jnp.float32)
    z_h = jnp.zeros((_RPT, _W), jnp.float32)
    W1p = jnp.pad(W1, ((0, 0), (0, _W - _H)))
    W2p = jnp.pad(W2, ((0, 0), (0, _W - _H)))

    degp = _deg_kernel()(dst_r, ones_d, z_d)
    dinv, hs1, offs = _prep_call(degp, x, W1p, batch)
    acc1 = _edge_kernel()(src_f, dst_r, hs1, z_h)
    hs2 = _mid_call(acc1, hs1, dinv, b1.reshape(1, _H),
                    gn_w.reshape(1, _H), gn_b.reshape(1, _H),
                    gn_ms.reshape(1, _H), ln_w.reshape(1, _H),
                    ln_b.reshape(1, _H), W2p)
    acc2 = _edge_kernel()(src_f, dst_r, hs2, z_h)
    out = _final_call(acc2, hs2, dinv, b2.reshape(1, _H),
                      gn_w.reshape(1, _H), gn_b.reshape(1, _H),
                      gn_ms.reshape(1, _H), ln_w.reshape(1, _H),
                      ln_b.reshape(1, _H),
                      jnp.pad(batch, (0, _NP - _N), constant_values=_B),
                      offs, fcW, fcb.reshape(1, _DOUT))
    return out
